# baseline (speedup 1.0000x reference)
import functools

import numpy as np
import jax
import jax.numpy as jnp
from jax import lax
from jax.experimental import pallas as pl
from jax.experimental.pallas import tpu as pltpu
from jax.experimental.pallas import tpu_sc as plsc

_B, _S, _D, _V = 1024, 200, 128, 16
_N = _B * _S
_NC, _NS = 2, 16
_NW = _NC * _NS
_BPW = _B // _NW
_PER_W = _N // _NW
_CH = 80
_STEPS = _PER_W // _CH
_PCH = 40
_NB = 5
_PF = 4


def _make_pe() -> np.ndarray:
    i = np.arange(_S, dtype=np.float64).reshape(-1, 1)
    j = np.power(10000.0, np.arange(0, _D, 2, dtype=np.float64) / _D)
    pe = np.zeros((_S, _D), np.float32)
    pe[:, 0::2] = np.sin(i / j)
    pe[:, 1::2] = np.cos(i / j)
    return pe


_PE = _make_pe()

_mesh = plsc.VectorSubcoreMesh(core_axis_name="c", subcore_axis_name="s")


@functools.partial(
    pl.kernel,
    out_type=jax.ShapeDtypeStruct((_N, _D), jnp.float32),
    mesh=_mesh,
    scratch_types=[
        pltpu.VMEM((_PCH, _D), jnp.float32),
        pltpu.VMEM((_PCH, _D), jnp.float32),
        pltpu.VMEM((_PCH, _D), jnp.float32),
        pltpu.VMEM((_PCH, _D), jnp.float32),
        pltpu.VMEM((_D,), jnp.float32),
        pltpu.VMEM_SHARED((_V * _S, _D), jnp.float32),
        pltpu.VMEM((_PER_W,), jnp.int32),
        pltpu.VMEM((_STEPS, _CH), jnp.int32),
        pltpu.VMEM((_CH, _D), jnp.float32),
        pltpu.VMEM((_CH, _D), jnp.float32),
        pltpu.VMEM((_CH, _D), jnp.float32),
        pltpu.VMEM((_CH, _D), jnp.float32),
        pltpu.VMEM((_CH, _D), jnp.float32),
        pltpu.SemaphoreType.DMA,
        pltpu.SemaphoreType.DMA,
        pltpu.SemaphoreType.DMA,
    ],
)
def _sc_kernel(emb_hbm, pe_hbm, x_hbm, out_hbm,
               peb0, peb1, cmb0, cmb1, trow_v, comb_sh, xv, idxv,
               buf0, buf1, buf2, buf3, buf4, xsem, gsem, wsem):
    cid = lax.axis_index("c")
    sid = lax.axis_index("s")
    wid = sid * _NC + cid

    xh = pltpu.async_copy(x_hbm.at[pl.ds(wid * _PER_W, _PER_W)], xv, xsem)

    pltpu.sync_copy(emb_hbm.at[sid], trow_v)
    tr = tuple(trow_v[pl.ds(c * 16, 16)] for c in range(_D // 16))

    pebs, cmbs = (peb0, peb1), (cmb0, cmb1)
    _NQ = _S // _PCH
    ph = [None] * _NQ
    ch = [None] * _NQ
    ph[0] = pltpu.async_copy(pe_hbm.at[pl.ds(0, _PCH)], pebs[0], gsem)
    for q in range(_NQ):
        if q + 1 < _NQ:
            ph[q + 1] = pltpu.async_copy(
                pe_hbm.at[pl.ds((q + 1) * _PCH, _PCH)], pebs[(q + 1) % 2], gsem)
        ph[q].wait()
        if q >= 2:
            ch[q - 2].wait()
        peb, cmb = pebs[q % 2], cmbs[q % 2]

        def build_rows(s4, tr, peb=peb, cmb=cmb):
            for u in range(4):
                s = s4 * 4 + u
                for c in range(_D // 16):
                    sl = pl.ds(c * 16, 16)
                    cmb[s, sl] = peb[s, sl] + tr[c]
            return tr

        lax.fori_loop(0, _PCH // 4, build_rows, tr)
        ch[q] = pltpu.async_copy(
            cmb, comb_sh.at[pl.ds(sid * _S + q * _PCH, _PCH)], wsem)

    for q in range(max(0, _NQ - 2), _NQ):
        ch[q].wait()
    xh.wait()
    plsc.subcore_barrier()

    base = wid * _PER_W
    lane = lax.iota(jnp.int32, 16)

    def idx_row(j):
        for c in range(_CH // 16):
            off = j * _CH + c * 16
            xi = xv[pl.ds(off, 16)]
            sv = lax.rem(off + lane, jnp.int32(_S))
            idxv[j, pl.ds(c * 16, 16)] = xi * _S + sv

    for j in range(_PF):
        idx_row(j)

    bufs = (buf0, buf1, buf2, buf3, buf4)
    gh = [None] * _STEPS
    wh = [None] * _STEPS
    for g in range(_STEPS + 1):
        if g < _STEPS:
            if g >= _NB:
                wh[g - _NB].wait()
            gh[g] = pltpu.async_copy(comb_sh.at[idxv.at[g]], bufs[g % _NB], gsem)
            if g + _PF < _STEPS:
                idx_row(g + _PF)
        if g >= 1:
            j = g - 1
            gh[j].wait()
            wh[j] = pltpu.async_copy(
                bufs[j % _NB], out_hbm.at[pl.ds(base + j * _CH, _CH)], wsem)
    for j in range(_STEPS - _NB, _STEPS):
        wh[j].wait()


def kernel(nucleo_emb, X):
    out = _sc_kernel(nucleo_emb, jnp.asarray(_PE), X.reshape(_N))
    return out.reshape(_B, _S, _D)


# speedup vs baseline: 1.0128x; 1.0128x over previous
import functools

import numpy as np
import jax
import jax.numpy as jnp
from jax import lax
from jax.experimental import pallas as pl
from jax.experimental.pallas import tpu as pltpu
from jax.experimental.pallas import tpu_sc as plsc

_B, _S, _D, _V = 1024, 200, 128, 16
_N = _B * _S
_NC, _NS = 2, 16
_NW = _NC * _NS
_BPW = _B // _NW
_PER_W = _N // _NW
_CH = 128
_STEPS = _PER_W // _CH
_PCH = 40
_NB = 4
_PF = 4


def _make_pe() -> np.ndarray:
    i = np.arange(_S, dtype=np.float64).reshape(-1, 1)
    j = np.power(10000.0, np.arange(0, _D, 2, dtype=np.float64) / _D)
    pe = np.zeros((_S, _D), np.float32)
    pe[:, 0::2] = np.sin(i / j)
    pe[:, 1::2] = np.cos(i / j)
    return pe


_PE = _make_pe()

_mesh = plsc.VectorSubcoreMesh(core_axis_name="c", subcore_axis_name="s")


@functools.partial(
    pl.kernel,
    out_type=jax.ShapeDtypeStruct((_N, _D), jnp.float32),
    mesh=_mesh,
    scratch_types=[
        pltpu.VMEM((4 * _PCH, _D), jnp.float32),
        pltpu.VMEM((_D,), jnp.float32),
        pltpu.VMEM_SHARED((_V * _S, _D), jnp.float32),
        pltpu.VMEM((_PER_W,), jnp.int32),
        pltpu.VMEM((_STEPS, _CH), jnp.int32),
        pltpu.VMEM((_NB * _CH, _D), jnp.float32),
        pltpu.SemaphoreType.DMA((3,)),
    ],
)
def _sc_kernel(emb_hbm, pe_hbm, x_hbm, out_hbm,
               work, trow_v, comb_sh, xv, idxv, ring, sems):
    xsem, gsem, wsem = sems.at[0], sems.at[1], sems.at[2]
    cid = lax.axis_index("c")
    sid = lax.axis_index("s")
    wid = sid * _NC + cid

    xh = pltpu.async_copy(x_hbm.at[pl.ds(wid * _PER_W, _PER_W)], xv, xsem)

    pltpu.sync_copy(emb_hbm.at[sid], trow_v)
    tr = tuple(trow_v[pl.ds(c * 16, 16)] for c in range(_D // 16))

    _NQ = _S // _PCH
    ph = [None] * _NQ
    ch = [None] * _NQ
    ph[0] = pltpu.async_copy(pe_hbm.at[pl.ds(0, _PCH)],
                             work.at[pl.ds(0, _PCH)], gsem)
    for q in range(_NQ):
        poff = (q % 2) * _PCH
        coff = 2 * _PCH + (q % 2) * _PCH
        if q + 1 < _NQ:
            ph[q + 1] = pltpu.async_copy(
                pe_hbm.at[pl.ds((q + 1) * _PCH, _PCH)],
                work.at[pl.ds(((q + 1) % 2) * _PCH, _PCH)], gsem)
        ph[q].wait()
        if q >= 2:
            ch[q - 2].wait()

        def build_rows(s4, tr, poff=poff, coff=coff):
            for u in range(4):
                s = s4 * 4 + u
                for c in range(_D // 16):
                    sl = pl.ds(c * 16, 16)
                    work[coff + s, sl] = work[poff + s, sl] + tr[c]
            return tr

        lax.fori_loop(0, _PCH // 4, build_rows, tr)
        ch[q] = pltpu.async_copy(
            work.at[pl.ds(coff, _PCH)],
            comb_sh.at[pl.ds(sid * _S + q * _PCH, _PCH)], wsem)

    for q in range(max(0, _NQ - 2), _NQ):
        ch[q].wait()
    xh.wait()
    plsc.subcore_barrier()

    base = wid * _PER_W
    lane = lax.iota(jnp.int32, 16)

    def idx_row(j):
        for c in range(_CH // 16):
            off = j * _CH + c * 16
            xi = xv[pl.ds(off, 16)]
            sv = lax.rem(off + lane, jnp.int32(_S))
            idxv[j, pl.ds(c * 16, 16)] = xi * _S + sv

    for j in range(_PF):
        idx_row(j)

    bufs = tuple(ring.at[pl.ds(b * _CH, _CH)] for b in range(_NB))
    gh = [None] * _STEPS
    wh = [None] * _STEPS
    for g in range(_STEPS + 1):
        if g < _STEPS:
            if g >= _NB:
                wh[g - _NB].wait()
            gh[g] = pltpu.async_copy(comb_sh.at[idxv.at[g]], bufs[g % _NB], gsem)
            if g + _PF < _STEPS:
                idx_row(g + _PF)
        if g >= 1:
            j = g - 1
            gh[j].wait()
            wh[j] = pltpu.async_copy(
                bufs[j % _NB], out_hbm.at[pl.ds(base + j * _CH, _CH)], wsem)
    for j in range(_STEPS - _NB, _STEPS):
        wh[j].wait()


def kernel(nucleo_emb, X):
    out = _sc_kernel(nucleo_emb, jnp.asarray(_PE), X.reshape(_N))
    return out.reshape(_B, _S, _D)
